# baseline (speedup 1.0000x reference)
import functools

import jax
import jax.numpy as jnp
from jax import lax
from jax.experimental import pallas as pl
from jax.experimental.pallas import tpu as pltpu
from jax.experimental.pallas import tpu_sc as plsc

N_NODES = 10000
N_EDGES = 320000
D_IN = 128
HIDDEN = 128
HEADS = 8
OUTC = HIDDEN // HEADS

NC = 2
NS = 16
NW = NC * NS
EPW = N_EDGES // NW
CHUNK = 80
NCHUNK = EPW // CHUNK
NPAD = 10240
ROWS_PER_TILE = NPAD // NS
ZROWS = 128


def _leaky(t):
    return jnp.where(t >= 0, t, 0.2 * t)


def _elu(x):
    return jnp.where(x > 0, x, jnp.exp(jnp.minimum(x, 0.0)) - 1.0)


def _in_body(emb, w1t, b1, x_o):
    x = jnp.dot(emb[...], w1t[...], preferred_element_type=jnp.float32,
                precision=lax.Precision.HIGHEST)
    x_o[...] = x + b1[...]


def _proj_body(x, wc, ps, pd, h_o, as_o, ad_o, am_o):
    h = jnp.dot(x[...], wc[...], preferred_element_type=jnp.float32,
                precision=lax.Precision.HIGHEST)
    a_s = jnp.dot(h, ps[...], preferred_element_type=jnp.float32,
                precision=lax.Precision.HIGHEST)
    a_d = jnp.dot(h, pd[...], preferred_element_type=jnp.float32,
                precision=lax.Precision.HIGHEST)
    m = jnp.max(a_s, axis=0, keepdims=True)
    h_o[...] = h
    as_o[...] = a_s
    ad_o[...] = a_d
    am_o[...] = _leaky(m + a_d)


def _norm_body(sp, dp, r, bc, x_o):
    sarr = sp[...]
    darr = dp[...]
    den = jnp.dot(darr[0, :N_NODES] + darr[1, :N_NODES], r[...],
                  preferred_element_type=jnp.float32,
                precision=lax.Precision.HIGHEST)
    x = (sarr[0, :N_NODES] + sarr[1, :N_NODES]) / (den + 1e-16) + bc[...]
    x_o[...] = _elu(x)


def _fin_body(x, w3, b3, sc_o):
    s = jnp.sum(x[...] * w3[...], axis=1, keepdims=True) + b3[...]
    sc_o[...] = jnp.broadcast_to(s, (N_NODES, 16))


_NF = jnp.float32
_in_call = pl.pallas_call(
    _in_body,
    out_shape=[jax.ShapeDtypeStruct((N_NODES, HIDDEN), _NF)],
)
_proj_call = pl.pallas_call(
    _proj_body,
    out_shape=[jax.ShapeDtypeStruct((N_NODES, HIDDEN), _NF),
               jax.ShapeDtypeStruct((N_NODES, 16), _NF),
               jax.ShapeDtypeStruct((N_NODES, 16), _NF),
               jax.ShapeDtypeStruct((N_NODES, 16), _NF)],
)
_norm_call = pl.pallas_call(
    _norm_body,
    out_shape=[jax.ShapeDtypeStruct((N_NODES, HIDDEN), _NF)],
)
_fin_call = pl.pallas_call(
    _fin_body,
    out_shape=[jax.ShapeDtypeStruct((N_NODES, 16), _NF)],
)


@functools.partial(
    pl.kernel,
    mesh=plsc.VectorSubcoreMesh(core_axis_name="c", subcore_axis_name="s"),
    compiler_params=pltpu.CompilerParams(use_tc_tiling_on_sc=False),
    out_type=[jax.ShapeDtypeStruct((NC, NPAD, HIDDEN), _NF),
              jax.ShapeDtypeStruct((NC, NPAD, 16), _NF)],
    scratch_types=[
        pltpu.VMEM((2, CHUNK), jnp.int32),
        pltpu.VMEM((2, CHUNK), jnp.int32),
        pltpu.VMEM((CHUNK, 16), _NF),
        pltpu.VMEM((CHUNK, 16), _NF),
        pltpu.VMEM((CHUNK, 16), _NF),
        pltpu.VMEM((CHUNK, 16), _NF),
        pltpu.VMEM((CHUNK, 16), _NF),
        pltpu.VMEM((CHUNK, 16), _NF),
        pltpu.VMEM((CHUNK, HIDDEN), _NF),
        pltpu.VMEM((CHUNK, HIDDEN), _NF),
        pltpu.VMEM((CHUNK, 16), _NF),
        pltpu.VMEM((CHUNK, 16), _NF),
        pltpu.VMEM_SHARED((NPAD, HIDDEN), _NF),
        pltpu.VMEM_SHARED((NPAD, 16), _NF),
    ] + [pltpu.SemaphoreType.DMA] * 10,
)
def _edge_call(edge_h, as_h, ad_h, am_h, h_h, out_s, out_d,
               idxA, idxB, asA, asB, adA, adB, amA, amB, hA, hB, exA, exB,
               acc_s, acc_d,
               sa0, sa1, sa2, sa3, sb0, sb1, sb2, sb3, six_a, six_b):
    c = lax.axis_index("c")
    s = lax.axis_index("s")
    wid = s * NC + c
    rbase = s * ROWS_PER_TILE
    sets = {
        0: (idxA, asA, adA, amA, hA, exA, (sa0, sa1, sa2, sa3), six_a),
        1: (idxB, asB, adB, amB, hB, exB, (sb0, sb1, sb2, sb3), six_b),
    }

    def zrow(i, carry):
        for k in range(HIDDEN // 16):
            hA[i, pl.ds(16 * k, 16)] = jnp.zeros((16,), _NF)
        exA[i] = jnp.zeros((16,), _NF)
        return carry

    lax.fori_loop(0, CHUNK, zrow, 0)

    def zcp(p, carry):
        r0 = rbase + p * CHUNK
        pltpu.sync_copy(hA, acc_s.at[pl.ds(r0, CHUNK)])
        pltpu.sync_copy(exA, acc_d.at[pl.ds(r0, CHUNK)])
        return carry

    lax.fori_loop(0, ROWS_PER_TILE // CHUNK, zcp, 0)
    plsc.subcore_barrier()

    mask8 = lax.iota(jnp.int32, 16) < 8

    def issue_gathers(b):
        idxv, asv, adv, amv, hv, _, sems, _ = sets[b]
        pltpu.async_copy(as_h.at[idxv.at[0]], asv, sems[0])
        pltpu.async_copy(ad_h.at[idxv.at[1]], adv, sems[1])
        pltpu.async_copy(am_h.at[idxv.at[1]], amv, sems[2])
        pltpu.async_copy(h_h.at[idxv.at[0]], hv, sems[3])

    def wait_gathers(b):
        idxv, asv, adv, amv, hv, _, sems, _ = sets[b]
        pltpu.make_async_copy(as_h.at[idxv.at[0]], asv, sems[0]).wait()
        pltpu.make_async_copy(ad_h.at[idxv.at[1]], adv, sems[1]).wait()
        pltpu.make_async_copy(am_h.at[idxv.at[1]], amv, sems[2]).wait()
        pltpu.make_async_copy(h_h.at[idxv.at[0]], hv, sems[3]).wait()

    def issue_idx(b, j):
        idxv, *_, sem = sets[b]
        pltpu.async_copy(edge_h.at[wid, j], idxv, sem)

    def wait_idx(b, j):
        idxv, *_, sem = sets[b]
        pltpu.make_async_copy(edge_h.at[wid, j], idxv, sem).wait()

    def compute_scatter(b):
        idxv, asv, adv, amv, hv, exv, _, _ = sets[b]

        def edge(e, ecarry):
            a = asv[e] + adv[e]
            ex = jnp.exp(_leaky(a) - amv[e])
            exm = jnp.where(mask8, ex, 0.0)
            exv[e] = exm
            for k in range(HEADS):
                w = exm[k]
                hv[e, pl.ds(16 * k, 16)] = hv[e, pl.ds(16 * k, 16)] * w
            return ecarry

        lax.fori_loop(0, CHUNK, edge, 0, unroll=4)
        pltpu.sync_copy(exv, acc_d.at[idxv.at[1]], add=True)
        pltpu.sync_copy(hv, acc_s.at[idxv.at[1]], add=True)

    pltpu.sync_copy(edge_h.at[wid, 0], idxA)
    issue_gathers(0)
    issue_idx(1, 1)

    def pipe(g, carry):
        j_a = 2 * g
        j_b = 2 * g + 1

        @pl.when(j_b < NCHUNK)
        def _():
            wait_idx(1, j_b)
            issue_gathers(1)

        wait_gathers(0)
        compute_scatter(0)

        @pl.when(j_a + 2 < NCHUNK)
        def _():
            issue_idx(0, j_a + 2)

        @pl.when(j_b < NCHUNK)
        def _():
            wait_gathers(1)
            compute_scatter(1)

        @pl.when(j_b + 2 < NCHUNK)
        def _():
            issue_idx(1, j_b + 2)

        @pl.when(j_a + 2 < NCHUNK)
        def _():
            wait_idx(0, j_a + 2)
            issue_gathers(0)

        return carry

    lax.fori_loop(0, (NCHUNK + 1) // 2, pipe, 0)
    plsc.subcore_barrier()

    def drain(p, carry):
        r0 = rbase + p * CHUNK
        pltpu.sync_copy(acc_s.at[pl.ds(r0, CHUNK)], hA)
        pltpu.sync_copy(hA, out_s.at[c, pl.ds(r0, CHUNK)])
        pltpu.sync_copy(acc_d.at[pl.ds(r0, CHUNK)], exA)
        pltpu.sync_copy(exA, out_d.at[c, pl.ds(r0, CHUNK)])
        return carry

    lax.fori_loop(0, ROWS_PER_TILE // CHUNK, drain, 0)


@functools.partial(
    pl.kernel,
    mesh=plsc.VectorSubcoreMesh(core_axis_name="c", subcore_axis_name="s"),
    compiler_params=pltpu.CompilerParams(use_tc_tiling_on_sc=False),
    out_type=[jax.ShapeDtypeStruct((256, 16), _NF)],
    scratch_types=[
        pltpu.VMEM((2, 128), jnp.int32),
        pltpu.VMEM((128, 16), _NF),
    ],
)
def _tgt_call(scores_h, tgt_h, out_h, tg_v, ot_v):
    c = lax.axis_index("c")
    s = lax.axis_index("s")

    @pl.when(jnp.logical_and(c == 0, s == 0))
    def _():
        pltpu.sync_copy(tgt_h, tg_v)
        for p in range(2):
            pltpu.sync_copy(scores_h.at[tg_v.at[p]], ot_v)
            pltpu.sync_copy(ot_v, out_h.at[pl.ds(128 * p, 128)])


def _expand_att(att):
    a = att.reshape(HEADS * OUTC).astype(jnp.float32)
    m = jnp.repeat(jnp.eye(HEADS, dtype=jnp.float32), OUTC, axis=0)
    p8 = m * a[:, None]
    return jnp.concatenate([p8, jnp.zeros((HIDDEN, 16 - HEADS), jnp.float32)], axis=1)


def kernel(word_embed_matrix, target_mask_list, graph_edge_list, W1, b1,
           Wc0, att_src0, att_dst0, bc0, Wc1, att_src1, att_dst1, bc1, W3, b3):
    edges = jnp.swapaxes(
        graph_edge_list.astype(jnp.int32).reshape(2, NW, NCHUNK, CHUNK),
        0, 1).swapaxes(1, 2)
    r_mat = jnp.repeat(jnp.eye(16, dtype=jnp.float32)[:, :HEADS], OUTC, axis=1)

    (x0,) = _in_call(word_embed_matrix, W1.T, b1.reshape(1, -1))
    wc_s = jnp.stack([Wc0, Wc1])
    ps_s = jnp.stack([_expand_att(att_src0), _expand_att(att_src1)])
    pd_s = jnp.stack([_expand_att(att_dst0), _expand_att(att_dst1)])
    bc_s = jnp.stack([bc0.reshape(1, -1), bc1.reshape(1, -1)])

    def body(x, ws):
        wc, ps, pd, bc = ws
        h, a_s, a_d, am = _proj_call(x, wc, ps, pd)
        s, d = _edge_call(edges, a_s, a_d, am, h)
        (xn,) = _norm_call(s, d, r_mat, bc)
        return xn, None

    x2, _ = lax.scan(body, x0, (wc_s, ps_s, pd_s, bc_s))
    (scores,) = _fin_call(x2, W3.reshape(1, -1), b3.reshape(1, 1))
    tgt = target_mask_list.reshape(2, 128).astype(jnp.int32)
    (out2,) = _tgt_call(scores, tgt)
    return out2[:, 0]


# speedup vs baseline: 1.4696x; 1.4696x over previous
import functools

import jax
import jax.numpy as jnp
from jax import lax
from jax.experimental import pallas as pl
from jax.experimental.pallas import tpu as pltpu
from jax.experimental.pallas import tpu_sc as plsc

N_NODES = 10000
N_EDGES = 320000
D_IN = 128
HIDDEN = 128
HEADS = 8
OUTC = HIDDEN // HEADS

NC = 2
NS = 16
NW = NC * NS
EPW = N_EDGES // NW
CHUNK = 80
NCHUNK = EPW // CHUNK
NPAD = 10240
ROWS_PER_TILE = NPAD // NS
ZROWS = 128


def _leaky(t):
    return jnp.where(t >= 0, t, 0.2 * t)


def _elu(x):
    return jnp.where(x > 0, x, jnp.exp(jnp.minimum(x, 0.0)) - 1.0)


def _in_body(emb, w1t, b1, x_o):
    x = jnp.dot(emb[...], w1t[...], preferred_element_type=jnp.float32,
                precision=lax.Precision.HIGHEST)
    x_o[...] = x + b1[...]


def _proj_body(x, wc, ps, pd, h_o, as_o, ad_o, am_o):
    h = jnp.dot(x[...], wc[...], preferred_element_type=jnp.float32,
                precision=lax.Precision.HIGHEST)
    a_s = jnp.dot(h, ps[...], preferred_element_type=jnp.float32,
                precision=lax.Precision.HIGHEST)
    a_d = jnp.dot(h, pd[...], preferred_element_type=jnp.float32,
                precision=lax.Precision.HIGHEST)
    m = jnp.max(a_s, axis=0, keepdims=True)
    h_o[...] = h
    as_o[...] = a_s
    ad_o[...] = a_d
    am_o[...] = _leaky(m + a_d)


def _norm_body(sp, dp, r, bc, x_o):
    sarr = sp[...]
    darr = dp[...]
    den = jnp.dot(darr[0, :N_NODES] + darr[1, :N_NODES], r[...],
                  preferred_element_type=jnp.float32,
                precision=lax.Precision.HIGHEST)
    x = (sarr[0, :N_NODES] + sarr[1, :N_NODES]) / (den + 1e-16) + bc[...]
    x_o[...] = _elu(x)


def _fin_body(x, w3, b3, sc_o):
    s = jnp.sum(x[...] * w3[...], axis=1, keepdims=True) + b3[...]
    sc_o[...] = jnp.broadcast_to(s, (N_NODES, 16))


_NF = jnp.float32
_in_call = pl.pallas_call(
    _in_body,
    out_shape=[jax.ShapeDtypeStruct((N_NODES, HIDDEN), _NF)],
)
_proj_call = pl.pallas_call(
    _proj_body,
    out_shape=[jax.ShapeDtypeStruct((N_NODES, HIDDEN), _NF),
               jax.ShapeDtypeStruct((N_NODES, 16), _NF),
               jax.ShapeDtypeStruct((N_NODES, 16), _NF),
               jax.ShapeDtypeStruct((N_NODES, 16), _NF)],
)
_norm_call = pl.pallas_call(
    _norm_body,
    out_shape=[jax.ShapeDtypeStruct((N_NODES, HIDDEN), _NF)],
)
_fin_call = pl.pallas_call(
    _fin_body,
    out_shape=[jax.ShapeDtypeStruct((N_NODES, 16), _NF)],
)


@functools.partial(
    pl.kernel,
    mesh=plsc.VectorSubcoreMesh(core_axis_name="c", subcore_axis_name="s"),
    compiler_params=pltpu.CompilerParams(use_tc_tiling_on_sc=False),
    out_type=[jax.ShapeDtypeStruct((NC, NPAD, HIDDEN), _NF),
              jax.ShapeDtypeStruct((NC, NPAD, 16), _NF)],
    scratch_types=[
        pltpu.VMEM((2, CHUNK), jnp.int32),
        pltpu.VMEM((2, CHUNK), jnp.int32),
        pltpu.VMEM((CHUNK, 16), _NF),
        pltpu.VMEM((CHUNK, 16), _NF),
        pltpu.VMEM((CHUNK, 16), _NF),
        pltpu.VMEM((CHUNK, 16), _NF),
        pltpu.VMEM((CHUNK, 16), _NF),
        pltpu.VMEM((CHUNK, 16), _NF),
        pltpu.VMEM((CHUNK, HIDDEN), _NF),
        pltpu.VMEM((CHUNK, HIDDEN), _NF),
        pltpu.VMEM((CHUNK, 16), _NF),
        pltpu.VMEM((CHUNK, 16), _NF),
        pltpu.VMEM_SHARED((NPAD, HIDDEN), _NF),
        pltpu.VMEM_SHARED((NPAD, 16), _NF),
    ] + [pltpu.SemaphoreType.DMA] * 10,
)
def _edge_call(edge_h, as_h, ad_h, am_h, h_h, out_s, out_d,
               idxA, idxB, asA, asB, adA, adB, amA, amB, hA, hB, exA, exB,
               acc_s, acc_d,
               sa0, sa1, sa2, sa3, sb0, sb1, sb2, sb3, six_a, six_b):
    c = lax.axis_index("c")
    s = lax.axis_index("s")
    wid = s * NC + c
    rbase = s * ROWS_PER_TILE
    sets = {
        0: (idxA, asA, adA, amA, hA, exA, (sa0, sa1, sa2, sa3), six_a),
        1: (idxB, asB, adB, amB, hB, exB, (sb0, sb1, sb2, sb3), six_b),
    }

    def zrow(i, carry):
        for k in range(HIDDEN // 16):
            hA[i, pl.ds(16 * k, 16)] = jnp.zeros((16,), _NF)
        exA[i] = jnp.zeros((16,), _NF)
        return carry

    lax.fori_loop(0, CHUNK, zrow, 0)

    def zcp(p, carry):
        r0 = rbase + p * CHUNK
        pltpu.sync_copy(hA, acc_s.at[pl.ds(r0, CHUNK)])
        pltpu.sync_copy(exA, acc_d.at[pl.ds(r0, CHUNK)])
        return carry

    lax.fori_loop(0, ROWS_PER_TILE // CHUNK, zcp, 0)
    plsc.subcore_barrier()

    mask8 = lax.iota(jnp.int32, 16) < 8

    def issue_gathers(b):
        idxv, asv, adv, amv, hv, _, sems, _ = sets[b]
        pltpu.async_copy(as_h.at[idxv.at[0]], asv, sems[0])
        pltpu.async_copy(ad_h.at[idxv.at[1]], adv, sems[1])
        pltpu.async_copy(am_h.at[idxv.at[1]], amv, sems[2])
        pltpu.async_copy(h_h.at[idxv.at[0]], hv, sems[3])

    def wait_gathers(b):
        idxv, asv, adv, amv, hv, _, sems, _ = sets[b]
        pltpu.make_async_copy(as_h.at[idxv.at[0]], asv, sems[0]).wait()
        pltpu.make_async_copy(ad_h.at[idxv.at[1]], adv, sems[1]).wait()
        pltpu.make_async_copy(am_h.at[idxv.at[1]], amv, sems[2]).wait()
        pltpu.make_async_copy(h_h.at[idxv.at[0]], hv, sems[3]).wait()

    def issue_idx(b, j):
        idxv, *_, sem = sets[b]
        pltpu.async_copy(edge_h.at[wid, j], idxv, sem)

    def wait_idx(b, j):
        idxv, *_, sem = sets[b]
        pltpu.make_async_copy(edge_h.at[wid, j], idxv, sem).wait()

    def compute_scatter(b):
        idxv, asv, adv, amv, hv, exv, _, _ = sets[b]

        def edge(e, ecarry):
            a = asv[e] + adv[e]
            ex = jnp.exp(_leaky(a) - amv[e])
            exm = jnp.where(mask8, ex, 0.0)
            exv[e] = exm
            for k in range(HEADS):
                w = exm[k]
                hv[e, pl.ds(16 * k, 16)] = hv[e, pl.ds(16 * k, 16)] * w
            return ecarry

        lax.fori_loop(0, CHUNK, edge, 0)
        pltpu.sync_copy(exv, acc_d.at[idxv.at[1]], add=True)
        pltpu.sync_copy(hv, acc_s.at[idxv.at[1]], add=True)

    pltpu.sync_copy(edge_h.at[wid, 0], idxA)
    issue_gathers(0)
    issue_idx(1, 1)

    def pipe(g, carry):
        j_a = 2 * g
        j_b = 2 * g + 1

        @pl.when(j_b < NCHUNK)
        def _():
            wait_idx(1, j_b)
            issue_gathers(1)

        wait_gathers(0)
        compute_scatter(0)

        @pl.when(j_a + 2 < NCHUNK)
        def _():
            issue_idx(0, j_a + 2)

        @pl.when(j_b < NCHUNK)
        def _():
            wait_gathers(1)
            compute_scatter(1)

        @pl.when(j_b + 2 < NCHUNK)
        def _():
            issue_idx(1, j_b + 2)

        @pl.when(j_a + 2 < NCHUNK)
        def _():
            wait_idx(0, j_a + 2)
            issue_gathers(0)

        return carry

    lax.fori_loop(0, (NCHUNK + 1) // 2, pipe, 0)
    plsc.subcore_barrier()

    def drain(p, carry):
        r0 = rbase + p * CHUNK
        pltpu.sync_copy(acc_s.at[pl.ds(r0, CHUNK)], hA)
        pltpu.sync_copy(hA, out_s.at[c, pl.ds(r0, CHUNK)])
        pltpu.sync_copy(acc_d.at[pl.ds(r0, CHUNK)], exA)
        pltpu.sync_copy(exA, out_d.at[c, pl.ds(r0, CHUNK)])
        return carry

    lax.fori_loop(0, ROWS_PER_TILE // CHUNK, drain, 0)


@functools.partial(
    pl.kernel,
    mesh=plsc.VectorSubcoreMesh(core_axis_name="c", subcore_axis_name="s"),
    compiler_params=pltpu.CompilerParams(use_tc_tiling_on_sc=False),
    out_type=[jax.ShapeDtypeStruct((256, 16), _NF)],
    scratch_types=[
        pltpu.VMEM((2, 128), jnp.int32),
        pltpu.VMEM((128, 16), _NF),
    ],
)
def _tgt_call(scores_h, tgt_h, out_h, tg_v, ot_v):
    c = lax.axis_index("c")
    s = lax.axis_index("s")

    @pl.when(jnp.logical_and(c == 0, s == 0))
    def _():
        pltpu.sync_copy(tgt_h, tg_v)
        for p in range(2):
            pltpu.sync_copy(scores_h.at[tg_v.at[p]], ot_v)
            pltpu.sync_copy(ot_v, out_h.at[pl.ds(128 * p, 128)])


def _expand_att(att):
    a = att.reshape(HEADS * OUTC).astype(jnp.float32)
    m = jnp.repeat(jnp.eye(HEADS, dtype=jnp.float32), OUTC, axis=0)
    p8 = m * a[:, None]
    return jnp.concatenate([p8, jnp.zeros((HIDDEN, 16 - HEADS), jnp.float32)], axis=1)


def kernel(word_embed_matrix, target_mask_list, graph_edge_list, W1, b1,
           Wc0, att_src0, att_dst0, bc0, Wc1, att_src1, att_dst1, bc1, W3, b3):
    edges = jnp.swapaxes(
        graph_edge_list.astype(jnp.int32).reshape(2, NW, NCHUNK, CHUNK),
        0, 1).swapaxes(1, 2)
    r_mat = jnp.repeat(jnp.eye(16, dtype=jnp.float32)[:, :HEADS], OUTC, axis=1)

    (x0,) = _in_call(word_embed_matrix, W1.T, b1.reshape(1, -1))
    wc_s = jnp.stack([Wc0, Wc1])
    ps_s = jnp.stack([_expand_att(att_src0), _expand_att(att_src1)])
    pd_s = jnp.stack([_expand_att(att_dst0), _expand_att(att_dst1)])
    bc_s = jnp.stack([bc0.reshape(1, -1), bc1.reshape(1, -1)])

    def body(x, ws):
        wc, ps, pd, bc = ws
        h, a_s, a_d, am = _proj_call(x, wc, ps, pd)
        s, d = _edge_call(edges, a_s, a_d, am, h)
        (xn,) = _norm_call(s, d, r_mat, bc)
        return xn, None

    x2, _ = lax.scan(body, x0, (wc_s, ps_s, pd_s, bc_s))
    (scores,) = _fin_call(x2, W3.reshape(1, -1), b3.reshape(1, 1))
    tgt = target_mask_list.reshape(2, 128).astype(jnp.int32)
    (out2,) = _tgt_call(scores, tgt)
    return out2[:, 0]


# speedup vs baseline: 1.6901x; 1.1500x over previous
import functools

import jax
import jax.numpy as jnp
from jax import lax
from jax.experimental import pallas as pl
from jax.experimental.pallas import tpu as pltpu
from jax.experimental.pallas import tpu_sc as plsc

N_NODES = 10000
N_EDGES = 320000
D_IN = 128
HIDDEN = 128
HEADS = 8
OUTC = HIDDEN // HEADS

NC = 2
NS = 16
NW = NC * NS
EPW = N_EDGES // NW
CHUNK = 80
NCHUNK = EPW // CHUNK
NPAD = 10240
ROWS_PER_TILE = NPAD // NS
ZROWS = 128


def _leaky(t):
    return jnp.where(t >= 0, t, 0.2 * t)


def _elu(x):
    return jnp.where(x > 0, x, jnp.exp(jnp.minimum(x, 0.0)) - 1.0)


def _in_body(emb, w1t, b1, x_o):
    x = jnp.dot(emb[...], w1t[...], preferred_element_type=jnp.float32,
                precision=lax.Precision.HIGHEST)
    x_o[...] = x + b1[...]


def _proj_body(x, wc, ps, pd, h_o, as_o, ad_o, am_o):
    h = jnp.dot(x[...], wc[...], preferred_element_type=jnp.float32,
                precision=lax.Precision.HIGHEST)
    a_s = jnp.dot(h, ps[...], preferred_element_type=jnp.float32,
                precision=lax.Precision.HIGHEST)
    a_d = jnp.dot(h, pd[...], preferred_element_type=jnp.float32,
                precision=lax.Precision.HIGHEST)
    m = jnp.max(a_s, axis=0, keepdims=True)
    h_o[...] = h
    as_o[...] = a_s
    ad_o[...] = a_d
    am_o[...] = _leaky(m + a_d)


def _norm_body(sp, dp, r, bc, x_o):
    sarr = sp[...]
    darr = dp[...]
    den = jnp.dot(darr[0, :N_NODES] + darr[1, :N_NODES], r[...],
                  preferred_element_type=jnp.float32,
                precision=lax.Precision.HIGHEST)
    x = (sarr[0, :N_NODES] + sarr[1, :N_NODES]) / (den + 1e-16) + bc[...]
    x_o[...] = _elu(x)


def _fin_body(x, w3, b3, sc_o):
    s = jnp.sum(x[...] * w3[...], axis=1, keepdims=True) + b3[...]
    sc_o[...] = jnp.broadcast_to(s, (N_NODES, 16))


_NF = jnp.float32
_in_call = pl.pallas_call(
    _in_body,
    out_shape=[jax.ShapeDtypeStruct((N_NODES, HIDDEN), _NF)],
)
_proj_call = pl.pallas_call(
    _proj_body,
    out_shape=[jax.ShapeDtypeStruct((N_NODES, HIDDEN), _NF),
               jax.ShapeDtypeStruct((N_NODES, 16), _NF),
               jax.ShapeDtypeStruct((N_NODES, 16), _NF),
               jax.ShapeDtypeStruct((N_NODES, 16), _NF)],
)
_norm_call = pl.pallas_call(
    _norm_body,
    out_shape=[jax.ShapeDtypeStruct((N_NODES, HIDDEN), _NF)],
)
_fin_call = pl.pallas_call(
    _fin_body,
    out_shape=[jax.ShapeDtypeStruct((N_NODES, 16), _NF)],
)


@functools.partial(
    pl.kernel,
    mesh=plsc.VectorSubcoreMesh(core_axis_name="c", subcore_axis_name="s"),
    compiler_params=pltpu.CompilerParams(use_tc_tiling_on_sc=False),
    out_type=[jax.ShapeDtypeStruct((NC, NPAD, HIDDEN), _NF),
              jax.ShapeDtypeStruct((NC, NPAD, 16), _NF)],
    scratch_types=[
        pltpu.VMEM((2, CHUNK), jnp.int32),
        pltpu.VMEM((2, CHUNK), jnp.int32),
        pltpu.VMEM((CHUNK, 16), _NF),
        pltpu.VMEM((CHUNK, 16), _NF),
        pltpu.VMEM((CHUNK, 16), _NF),
        pltpu.VMEM((CHUNK, 16), _NF),
        pltpu.VMEM((CHUNK, 16), _NF),
        pltpu.VMEM((CHUNK, 16), _NF),
        pltpu.VMEM((CHUNK, HIDDEN), _NF),
        pltpu.VMEM((CHUNK, HIDDEN), _NF),
        pltpu.VMEM((CHUNK, 16), _NF),
        pltpu.VMEM((CHUNK, 16), _NF),
        pltpu.VMEM_SHARED((NPAD, HIDDEN), _NF),
        pltpu.VMEM_SHARED((NPAD, 16), _NF),
    ] + [pltpu.SemaphoreType.DMA] * 14,
)
def _edge_call(edge_h, as_h, ad_h, am_h, h_h, out_s, out_d,
               idxA, idxB, asA, asB, adA, adB, amA, amB, hA, hB, exA, exB,
               acc_s, acc_d,
               sa0, sa1, sa2, sa3, sb0, sb1, sb2, sb3, six_a, six_b,
               ssc_a0, ssc_a1, ssc_b0, ssc_b1):
    c = lax.axis_index("c")
    s = lax.axis_index("s")
    wid = s * NC + c
    rbase = s * ROWS_PER_TILE
    sets = {
        0: (idxA, asA, adA, amA, hA, exA, (sa0, sa1, sa2, sa3), six_a,
            (ssc_a0, ssc_a1)),
        1: (idxB, asB, adB, amB, hB, exB, (sb0, sb1, sb2, sb3), six_b,
            (ssc_b0, ssc_b1)),
    }

    def zrow(i, carry):
        for k in range(HIDDEN // 16):
            hA[i, pl.ds(16 * k, 16)] = jnp.zeros((16,), _NF)
        exA[i] = jnp.zeros((16,), _NF)
        return carry

    lax.fori_loop(0, CHUNK, zrow, 0)

    def zcp(p, carry):
        r0 = rbase + p * CHUNK
        pltpu.sync_copy(hA, acc_s.at[pl.ds(r0, CHUNK)])
        pltpu.sync_copy(exA, acc_d.at[pl.ds(r0, CHUNK)])
        return carry

    lax.fori_loop(0, ROWS_PER_TILE // CHUNK, zcp, 0)
    plsc.subcore_barrier()

    mask8 = lax.iota(jnp.int32, 16) < 8

    def issue_gathers(b):
        idxv, asv, adv, amv, hv, _, sems, _, _ = sets[b]
        pltpu.async_copy(as_h.at[idxv.at[0]], asv, sems[0])
        pltpu.async_copy(ad_h.at[idxv.at[1]], adv, sems[1])
        pltpu.async_copy(am_h.at[idxv.at[1]], amv, sems[2])
        pltpu.async_copy(h_h.at[idxv.at[0]], hv, sems[3])

    def wait_gathers(b):
        idxv, asv, adv, amv, hv, _, sems, _, _ = sets[b]
        pltpu.make_async_copy(as_h.at[idxv.at[0]], asv, sems[0]).wait()
        pltpu.make_async_copy(ad_h.at[idxv.at[1]], adv, sems[1]).wait()
        pltpu.make_async_copy(am_h.at[idxv.at[1]], amv, sems[2]).wait()
        pltpu.make_async_copy(h_h.at[idxv.at[0]], hv, sems[3]).wait()

    def issue_idx(b, j):
        idxv = sets[b][0]
        sem = sets[b][7]
        pltpu.async_copy(edge_h.at[wid, j], idxv, sem)

    def wait_idx(b, j):
        idxv = sets[b][0]
        sem = sets[b][7]
        pltpu.make_async_copy(edge_h.at[wid, j], idxv, sem).wait()

    def compute_scatter(b):
        idxv, asv, adv, amv, hv, exv, _, _, scs = sets[b]

        def edge(e, ecarry):
            a = asv[e] + adv[e]
            ex = jnp.exp(_leaky(a) - amv[e])
            exm = jnp.where(mask8, ex, 0.0)
            exv[e] = exm
            for k in range(HEADS):
                w = exm[k]
                hv[e, pl.ds(16 * k, 16)] = hv[e, pl.ds(16 * k, 16)] * w
            return ecarry

        lax.fori_loop(0, CHUNK, edge, 0)
        pltpu.async_copy(exv, acc_d.at[idxv.at[1]], scs[0], add=True)
        pltpu.async_copy(hv, acc_s.at[idxv.at[1]], scs[1], add=True)

    pltpu.sync_copy(edge_h.at[wid, 0], idxA)
    issue_gathers(0)
    issue_idx(1, 1)

    def wait_scatter(b):
        idxv, _, _, _, hv, exv, _, _, scs = sets[b]
        pltpu.make_async_copy(exv, acc_d.at[idxv.at[1]], scs[0]).wait()
        pltpu.make_async_copy(hv, acc_s.at[idxv.at[1]], scs[1]).wait()

    def pipe(g, carry):
        j_a = 2 * g
        j_b = 2 * g + 1

        @pl.when(g > 0)
        def _():
            wait_scatter(1)

        @pl.when(j_b < NCHUNK)
        def _():
            wait_idx(1, j_b)
            issue_gathers(1)

        wait_gathers(0)
        compute_scatter(0)

        @pl.when(j_a + 2 < NCHUNK)
        def _():
            issue_idx(0, j_a + 2)

        @pl.when(j_b < NCHUNK)
        def _():
            wait_gathers(1)
            compute_scatter(1)

        @pl.when(j_b + 2 < NCHUNK)
        def _():
            issue_idx(1, j_b + 2)

        @pl.when(j_a + 2 < NCHUNK)
        def _():
            wait_scatter(0)
            wait_idx(0, j_a + 2)
            issue_gathers(0)

        return carry

    lax.fori_loop(0, (NCHUNK + 1) // 2, pipe, 0)
    wait_scatter(0)
    plsc.subcore_barrier()

    def drain(p, carry):
        r0 = rbase + p * CHUNK
        pltpu.sync_copy(acc_s.at[pl.ds(r0, CHUNK)], hA)
        pltpu.sync_copy(hA, out_s.at[c, pl.ds(r0, CHUNK)])
        pltpu.sync_copy(acc_d.at[pl.ds(r0, CHUNK)], exA)
        pltpu.sync_copy(exA, out_d.at[c, pl.ds(r0, CHUNK)])
        return carry

    lax.fori_loop(0, ROWS_PER_TILE // CHUNK, drain, 0)


@functools.partial(
    pl.kernel,
    mesh=plsc.VectorSubcoreMesh(core_axis_name="c", subcore_axis_name="s"),
    compiler_params=pltpu.CompilerParams(use_tc_tiling_on_sc=False),
    out_type=[jax.ShapeDtypeStruct((256, 16), _NF)],
    scratch_types=[
        pltpu.VMEM((2, 128), jnp.int32),
        pltpu.VMEM((128, 16), _NF),
    ],
)
def _tgt_call(scores_h, tgt_h, out_h, tg_v, ot_v):
    c = lax.axis_index("c")
    s = lax.axis_index("s")

    @pl.when(jnp.logical_and(c == 0, s == 0))
    def _():
        pltpu.sync_copy(tgt_h, tg_v)
        for p in range(2):
            pltpu.sync_copy(scores_h.at[tg_v.at[p]], ot_v)
            pltpu.sync_copy(ot_v, out_h.at[pl.ds(128 * p, 128)])


def _expand_att(att):
    a = att.reshape(HEADS * OUTC).astype(jnp.float32)
    m = jnp.repeat(jnp.eye(HEADS, dtype=jnp.float32), OUTC, axis=0)
    p8 = m * a[:, None]
    return jnp.concatenate([p8, jnp.zeros((HIDDEN, 16 - HEADS), jnp.float32)], axis=1)


def kernel(word_embed_matrix, target_mask_list, graph_edge_list, W1, b1,
           Wc0, att_src0, att_dst0, bc0, Wc1, att_src1, att_dst1, bc1, W3, b3):
    edges = jnp.swapaxes(
        graph_edge_list.astype(jnp.int32).reshape(2, NW, NCHUNK, CHUNK),
        0, 1).swapaxes(1, 2)
    r_mat = jnp.repeat(jnp.eye(16, dtype=jnp.float32)[:, :HEADS], OUTC, axis=1)

    (x0,) = _in_call(word_embed_matrix, W1.T, b1.reshape(1, -1))
    wc_s = jnp.stack([Wc0, Wc1])
    ps_s = jnp.stack([_expand_att(att_src0), _expand_att(att_src1)])
    pd_s = jnp.stack([_expand_att(att_dst0), _expand_att(att_dst1)])
    bc_s = jnp.stack([bc0.reshape(1, -1), bc1.reshape(1, -1)])

    def body(x, ws):
        wc, ps, pd, bc = ws
        h, a_s, a_d, am = _proj_call(x, wc, ps, pd)
        s, d = _edge_call(edges, a_s, a_d, am, h)
        (xn,) = _norm_call(s, d, r_mat, bc)
        return xn, None

    x2, _ = lax.scan(body, x0, (wc_s, ps_s, pd_s, bc_s))
    (scores,) = _fin_call(x2, W3.reshape(1, -1), b3.reshape(1, 1))
    tgt = target_mask_list.reshape(2, 128).astype(jnp.int32)
    (out2,) = _tgt_call(scores, tgt)
    return out2[:, 0]


# speedup vs baseline: 1.7400x; 1.0295x over previous
import functools

import jax
import jax.numpy as jnp
from jax import lax
from jax.experimental import pallas as pl
from jax.experimental.pallas import tpu as pltpu
from jax.experimental.pallas import tpu_sc as plsc

N_NODES = 10000
N_EDGES = 320000
D_IN = 128
HIDDEN = 128
HEADS = 8
OUTC = HIDDEN // HEADS

NC = 2
NS = 16
NW = NC * NS
EPW = N_EDGES // NW
CHUNK = 80
NCHUNK = EPW // CHUNK
NPAD = 10240
ROWS_PER_TILE = NPAD // NS
ZROWS = 128


def _leaky(t):
    return jnp.where(t >= 0, t, 0.2 * t)


def _elu(x):
    return jnp.where(x > 0, x, jnp.exp(jnp.minimum(x, 0.0)) - 1.0)


def _in_body(emb, w1t, b1, x_o):
    x = jnp.dot(emb[...], w1t[...], preferred_element_type=jnp.float32,
                precision=lax.Precision.HIGHEST)
    x_o[...] = x + b1[...]


def _proj_body(x, wc, ps, pd, h_o, as_o, ad_o):
    h = jnp.dot(x[...], wc[...], preferred_element_type=jnp.float32,
                precision=lax.Precision.HIGHEST)
    a_s = jnp.dot(h, ps[...], preferred_element_type=jnp.float32,
                precision=lax.Precision.HIGHEST)
    a_d = jnp.dot(h, pd[...], preferred_element_type=jnp.float32,
                precision=lax.Precision.HIGHEST)
    m = jnp.max(a_s, axis=0, keepdims=True)
    am = _leaky(m + a_d)
    h_o[...] = h
    as_o[...] = a_s
    ad_o[...] = jnp.concatenate([a_d[:, :8], am[:, :8]], axis=1)


def _norm_body(sp, dp, r, bc, x_o):
    sarr = sp[...]
    darr = dp[...]
    den = jnp.dot(darr[0, :N_NODES] + darr[1, :N_NODES], r[...],
                  preferred_element_type=jnp.float32,
                precision=lax.Precision.HIGHEST)
    x = (sarr[0, :N_NODES] + sarr[1, :N_NODES]) / (den + 1e-16) + bc[...]
    x_o[...] = _elu(x)


def _fin_body(x, w3, b3, sc_o):
    s = jnp.sum(x[...] * w3[...], axis=1, keepdims=True) + b3[...]
    sc_o[...] = jnp.broadcast_to(s, (N_NODES, 16))


_NF = jnp.float32
_in_call = pl.pallas_call(
    _in_body,
    out_shape=[jax.ShapeDtypeStruct((N_NODES, HIDDEN), _NF)],
)
_proj_call = pl.pallas_call(
    _proj_body,
    out_shape=[jax.ShapeDtypeStruct((N_NODES, HIDDEN), _NF),
               jax.ShapeDtypeStruct((N_NODES, 16), _NF),
               jax.ShapeDtypeStruct((N_NODES, 16), _NF)],
)
_norm_call = pl.pallas_call(
    _norm_body,
    out_shape=[jax.ShapeDtypeStruct((N_NODES, HIDDEN), _NF)],
)
_fin_call = pl.pallas_call(
    _fin_body,
    out_shape=[jax.ShapeDtypeStruct((N_NODES, 16), _NF)],
)


@functools.partial(
    pl.kernel,
    mesh=plsc.VectorSubcoreMesh(core_axis_name="c", subcore_axis_name="s"),
    compiler_params=pltpu.CompilerParams(use_tc_tiling_on_sc=False),
    out_type=[jax.ShapeDtypeStruct((NC, NPAD, HIDDEN), _NF),
              jax.ShapeDtypeStruct((NC, NPAD, 16), _NF)],
    scratch_types=[
        pltpu.VMEM((2, CHUNK), jnp.int32),
        pltpu.VMEM((2, CHUNK), jnp.int32),
        pltpu.VMEM((CHUNK, 16), _NF),
        pltpu.VMEM((CHUNK, 16), _NF),
        pltpu.VMEM((CHUNK, 16), _NF),
        pltpu.VMEM((CHUNK, 16), _NF),
        pltpu.VMEM((CHUNK, HIDDEN), _NF),
        pltpu.VMEM((CHUNK, HIDDEN), _NF),
        pltpu.VMEM((CHUNK, 16), _NF),
        pltpu.VMEM((CHUNK, 16), _NF),
        pltpu.VMEM_SHARED((NPAD, HIDDEN), _NF),
        pltpu.VMEM_SHARED((NPAD, 16), _NF),
    ] + [pltpu.SemaphoreType.DMA] * 12,
)
def _edge_call(edge_h, as_h, ad_h, h_h, out_s, out_d,
               idxA, idxB, asA, asB, adA, adB, hA, hB, exA, exB,
               acc_s, acc_d,
               sa0, sa1, sa2, sb0, sb1, sb2, six_a, six_b,
               ssc_a0, ssc_a1, ssc_b0, ssc_b1):
    c = lax.axis_index("c")
    s = lax.axis_index("s")
    wid = s * NC + c
    rbase = s * ROWS_PER_TILE
    sets = {
        0: (idxA, asA, adA, hA, exA, (sa0, sa1, sa2), six_a,
            (ssc_a0, ssc_a1)),
        1: (idxB, asB, adB, hB, exB, (sb0, sb1, sb2), six_b,
            (ssc_b0, ssc_b1)),
    }

    def zrow(i, carry):
        for k in range(HIDDEN // 16):
            hA[i, pl.ds(16 * k, 16)] = jnp.zeros((16,), _NF)
        exA[i] = jnp.zeros((16,), _NF)
        return carry

    lax.fori_loop(0, CHUNK, zrow, 0)

    def zcp(p, carry):
        r0 = rbase + p * CHUNK
        pltpu.sync_copy(hA, acc_s.at[pl.ds(r0, CHUNK)])
        pltpu.sync_copy(exA, acc_d.at[pl.ds(r0, CHUNK)])
        return carry

    lax.fori_loop(0, ROWS_PER_TILE // CHUNK, zcp, 0)
    plsc.subcore_barrier()

    mask8 = lax.iota(jnp.int32, 16) < 8

    def issue_gathers(b):
        idxv, asv, adv, hv, _, sems, _, _ = sets[b]
        pltpu.async_copy(as_h.at[idxv.at[0]], asv, sems[0])
        pltpu.async_copy(ad_h.at[idxv.at[1]], adv, sems[1])
        pltpu.async_copy(h_h.at[idxv.at[0]], hv, sems[2])

    def wait_gathers(b):
        idxv, asv, adv, hv, _, sems, _, _ = sets[b]
        pltpu.make_async_copy(as_h.at[idxv.at[0]], asv, sems[0]).wait()
        pltpu.make_async_copy(ad_h.at[idxv.at[1]], adv, sems[1]).wait()
        pltpu.make_async_copy(h_h.at[idxv.at[0]], hv, sems[2]).wait()

    def issue_idx(b, j):
        idxv = sets[b][0]
        sem = sets[b][6]
        pltpu.async_copy(edge_h.at[wid, j], idxv, sem)

    def wait_idx(b, j):
        idxv = sets[b][0]
        sem = sets[b][6]
        pltpu.make_async_copy(edge_h.at[wid, j], idxv, sem).wait()

    _shift_idx = (lax.iota(jnp.int32, 16) + 8) % 16

    def compute_scatter(b):
        idxv, asv, adv, hv, exv, _, _, scs = sets[b]

        def edge(e, ecarry):
            adamv = adv[e]
            a = asv[e] + adamv
            sh = lax.gather(
                adamv, _shift_idx[:, None],
                dimension_numbers=lax.GatherDimensionNumbers(
                    offset_dims=(), collapsed_slice_dims=(0,),
                    start_index_map=(0,)),
                slice_sizes=(1,),
                mode=lax.GatherScatterMode.PROMISE_IN_BOUNDS)
            ex = jnp.exp(_leaky(a) - sh)
            exm = jnp.where(mask8, ex, 0.0)
            exv[e] = exm
            for k in range(HEADS):
                w = exm[k]
                hv[e, pl.ds(16 * k, 16)] = hv[e, pl.ds(16 * k, 16)] * w
            return ecarry

        lax.fori_loop(0, CHUNK, edge, 0)
        pltpu.async_copy(exv, acc_d.at[idxv.at[1]], scs[0], add=True)
        pltpu.async_copy(hv, acc_s.at[idxv.at[1]], scs[1], add=True)

    pltpu.sync_copy(edge_h.at[wid, 0], idxA)
    issue_gathers(0)
    issue_idx(1, 1)

    def wait_scatter(b):
        idxv, _, _, hv, exv, _, _, scs = sets[b]
        pltpu.make_async_copy(exv, acc_d.at[idxv.at[1]], scs[0]).wait()
        pltpu.make_async_copy(hv, acc_s.at[idxv.at[1]], scs[1]).wait()

    def pipe(g, carry):
        j_a = 2 * g
        j_b = 2 * g + 1

        @pl.when(g > 0)
        def _():
            wait_scatter(1)

        @pl.when(j_b < NCHUNK)
        def _():
            wait_idx(1, j_b)
            issue_gathers(1)

        wait_gathers(0)
        compute_scatter(0)

        @pl.when(j_a + 2 < NCHUNK)
        def _():
            issue_idx(0, j_a + 2)

        @pl.when(j_b < NCHUNK)
        def _():
            wait_gathers(1)
            compute_scatter(1)

        @pl.when(j_b + 2 < NCHUNK)
        def _():
            issue_idx(1, j_b + 2)

        @pl.when(j_a + 2 < NCHUNK)
        def _():
            wait_scatter(0)
            wait_idx(0, j_a + 2)
            issue_gathers(0)

        return carry

    lax.fori_loop(0, (NCHUNK + 1) // 2, pipe, 0)
    wait_scatter(0)
    plsc.subcore_barrier()

    def drain(p, carry):
        r0 = rbase + p * CHUNK
        pltpu.sync_copy(acc_s.at[pl.ds(r0, CHUNK)], hA)
        pltpu.sync_copy(hA, out_s.at[c, pl.ds(r0, CHUNK)])
        pltpu.sync_copy(acc_d.at[pl.ds(r0, CHUNK)], exA)
        pltpu.sync_copy(exA, out_d.at[c, pl.ds(r0, CHUNK)])
        return carry

    lax.fori_loop(0, ROWS_PER_TILE // CHUNK, drain, 0)


@functools.partial(
    pl.kernel,
    mesh=plsc.VectorSubcoreMesh(core_axis_name="c", subcore_axis_name="s"),
    compiler_params=pltpu.CompilerParams(use_tc_tiling_on_sc=False),
    out_type=[jax.ShapeDtypeStruct((256, 16), _NF)],
    scratch_types=[
        pltpu.VMEM((2, 128), jnp.int32),
        pltpu.VMEM((128, 16), _NF),
    ],
)
def _tgt_call(scores_h, tgt_h, out_h, tg_v, ot_v):
    c = lax.axis_index("c")
    s = lax.axis_index("s")

    @pl.when(jnp.logical_and(c == 0, s == 0))
    def _():
        pltpu.sync_copy(tgt_h, tg_v)
        for p in range(2):
            pltpu.sync_copy(scores_h.at[tg_v.at[p]], ot_v)
            pltpu.sync_copy(ot_v, out_h.at[pl.ds(128 * p, 128)])


def _expand_att(att):
    a = att.reshape(HEADS * OUTC).astype(jnp.float32)
    m = jnp.repeat(jnp.eye(HEADS, dtype=jnp.float32), OUTC, axis=0)
    p8 = m * a[:, None]
    return jnp.concatenate([p8, jnp.zeros((HIDDEN, 16 - HEADS), jnp.float32)], axis=1)


def kernel(word_embed_matrix, target_mask_list, graph_edge_list, W1, b1,
           Wc0, att_src0, att_dst0, bc0, Wc1, att_src1, att_dst1, bc1, W3, b3):
    edges = jnp.swapaxes(
        graph_edge_list.astype(jnp.int32).reshape(2, NW, NCHUNK, CHUNK),
        0, 1).swapaxes(1, 2)
    r_mat = jnp.repeat(jnp.eye(16, dtype=jnp.float32)[:, :HEADS], OUTC, axis=1)

    (x0,) = _in_call(word_embed_matrix, W1.T, b1.reshape(1, -1))
    wc_s = jnp.stack([Wc0, Wc1])
    ps_s = jnp.stack([_expand_att(att_src0), _expand_att(att_src1)])
    pd_s = jnp.stack([_expand_att(att_dst0), _expand_att(att_dst1)])
    bc_s = jnp.stack([bc0.reshape(1, -1), bc1.reshape(1, -1)])

    def body(x, ws):
        wc, ps, pd, bc = ws
        h, a_s, adam = _proj_call(x, wc, ps, pd)
        s, d = _edge_call(edges, a_s, adam, h)
        (xn,) = _norm_call(s, d, r_mat, bc)
        return xn, None

    x2, _ = lax.scan(body, x0, (wc_s, ps_s, pd_s, bc_s))
    (scores,) = _fin_call(x2, W3.reshape(1, -1), b3.reshape(1, 1))
    tgt = target_mask_list.reshape(2, 128).astype(jnp.int32)
    (out2,) = _tgt_call(scores, tgt)
    return out2[:, 0]


# speedup vs baseline: 1.7472x; 1.0042x over previous
import functools

import jax
import jax.numpy as jnp
from jax import lax
from jax.experimental import pallas as pl
from jax.experimental.pallas import tpu as pltpu
from jax.experimental.pallas import tpu_sc as plsc

N_NODES = 10000
N_EDGES = 320000
D_IN = 128
HIDDEN = 128
HEADS = 8
OUTC = HIDDEN // HEADS

NC = 2
NS = 16
NW = NC * NS
EPW = N_EDGES // NW
CHUNK = 80
NCHUNK = EPW // CHUNK
NPAD = 10240
ROWS_PER_TILE = NPAD // NS
ZROWS = 128


def _leaky(t):
    return jnp.where(t >= 0, t, 0.2 * t)


def _elu(x):
    return jnp.where(x > 0, x, jnp.exp(jnp.minimum(x, 0.0)) - 1.0)


def _proj_tail(x, wc, ps, pd, h_o, as_o, ad_o):
    h = jnp.dot(x, wc[...], preferred_element_type=jnp.float32,
                precision=lax.Precision.HIGHEST)
    a_s = jnp.dot(h, ps[...], preferred_element_type=jnp.float32,
                  precision=lax.Precision.HIGHEST)
    a_d = jnp.dot(h, pd[...], preferred_element_type=jnp.float32,
                  precision=lax.Precision.HIGHEST)
    m = jnp.max(a_s, axis=0, keepdims=True)
    am = _leaky(m + a_d)
    h_o[...] = h
    as_o[...] = a_s
    ad_o[...] = jnp.concatenate([a_d[:, :8], am[:, :8]], axis=1)


def _norm_x(sp, dp, r, bc):
    sarr = sp[...]
    darr = dp[...]
    den = jnp.dot(darr[0, :N_NODES] + darr[1, :N_NODES], r[...],
                  preferred_element_type=jnp.float32,
                  precision=lax.Precision.HIGHEST)
    x = (sarr[0, :N_NODES] + sarr[1, :N_NODES]) / (den + 1e-16) + bc[...]
    return _elu(x)


def _inproj_body(emb, w1t, b1, wc, ps, pd, h_o, as_o, ad_o):
    x = jnp.dot(emb[...], w1t[...], preferred_element_type=jnp.float32,
                precision=lax.Precision.HIGHEST) + b1[...]
    _proj_tail(x, wc, ps, pd, h_o, as_o, ad_o)


def _norm_body(sp, dp, r, bc, x_o):
    x_o[...] = _norm_x(sp, dp, r, bc)


def _proj_body(x, wc, ps, pd, h_o, as_o, ad_o):
    _proj_tail(x[...], wc, ps, pd, h_o, as_o, ad_o)


def _normfin_body(sp, dp, r, bc, w3, b3, sc_o):
    x = _norm_x(sp, dp, r, bc)
    s = jnp.sum(x * w3[...], axis=1, keepdims=True) + b3[...]
    sc_o[...] = jnp.broadcast_to(s, (N_NODES, 16))


_NF = jnp.float32
_PROJ_OUT = [jax.ShapeDtypeStruct((N_NODES, HIDDEN), _NF),
             jax.ShapeDtypeStruct((N_NODES, 16), _NF),
             jax.ShapeDtypeStruct((N_NODES, 16), _NF)]
_inproj_call = pl.pallas_call(_inproj_body, out_shape=_PROJ_OUT)
_norm_call = pl.pallas_call(
    _norm_body, out_shape=[jax.ShapeDtypeStruct((N_NODES, HIDDEN), _NF)])
_proj_call = pl.pallas_call(_proj_body, out_shape=_PROJ_OUT)
_normfin_call = pl.pallas_call(
    _normfin_body, out_shape=[jax.ShapeDtypeStruct((N_NODES, 16), _NF)])


@functools.partial(
    pl.kernel,
    mesh=plsc.VectorSubcoreMesh(core_axis_name="c", subcore_axis_name="s"),
    compiler_params=pltpu.CompilerParams(use_tc_tiling_on_sc=False),
    out_type=[jax.ShapeDtypeStruct((NC, NPAD, HIDDEN), _NF),
              jax.ShapeDtypeStruct((NC, NPAD, 16), _NF)],
    scratch_types=[
        pltpu.VMEM((2, CHUNK), jnp.int32),
        pltpu.VMEM((2, CHUNK), jnp.int32),
        pltpu.VMEM((CHUNK, 16), _NF),
        pltpu.VMEM((CHUNK, 16), _NF),
        pltpu.VMEM((CHUNK, 16), _NF),
        pltpu.VMEM((CHUNK, 16), _NF),
        pltpu.VMEM((CHUNK, HIDDEN), _NF),
        pltpu.VMEM((CHUNK, HIDDEN), _NF),
        pltpu.VMEM((CHUNK, 16), _NF),
        pltpu.VMEM((CHUNK, 16), _NF),
        pltpu.VMEM_SHARED((NPAD, HIDDEN), _NF),
        pltpu.VMEM_SHARED((NPAD, 16), _NF),
    ] + [pltpu.SemaphoreType.DMA] * 12,
)
def _edge_call(edge_h, as_h, ad_h, h_h, out_s, out_d,
               idxA, idxB, asA, asB, adA, adB, hA, hB, exA, exB,
               acc_s, acc_d,
               sa0, sa1, sa2, sb0, sb1, sb2, six_a, six_b,
               ssc_a0, ssc_a1, ssc_b0, ssc_b1):
    c = lax.axis_index("c")
    s = lax.axis_index("s")
    wid = s * NC + c
    rbase = s * ROWS_PER_TILE
    sets = {
        0: (idxA, asA, adA, hA, exA, (sa0, sa1, sa2), six_a,
            (ssc_a0, ssc_a1)),
        1: (idxB, asB, adB, hB, exB, (sb0, sb1, sb2), six_b,
            (ssc_b0, ssc_b1)),
    }

    def zrow(i, carry):
        for k in range(HIDDEN // 16):
            hA[i, pl.ds(16 * k, 16)] = jnp.zeros((16,), _NF)
        exA[i] = jnp.zeros((16,), _NF)
        return carry

    lax.fori_loop(0, CHUNK, zrow, 0)

    def zcp(p, carry):
        r0 = rbase + p * CHUNK
        pltpu.sync_copy(hA, acc_s.at[pl.ds(r0, CHUNK)])
        pltpu.sync_copy(exA, acc_d.at[pl.ds(r0, CHUNK)])
        return carry

    lax.fori_loop(0, ROWS_PER_TILE // CHUNK, zcp, 0)
    plsc.subcore_barrier()

    mask8 = lax.iota(jnp.int32, 16) < 8

    def issue_gathers(b):
        idxv, asv, adv, hv, _, sems, _, _ = sets[b]
        pltpu.async_copy(as_h.at[idxv.at[0]], asv, sems[0])
        pltpu.async_copy(ad_h.at[idxv.at[1]], adv, sems[1])
        pltpu.async_copy(h_h.at[idxv.at[0]], hv, sems[2])

    def wait_gathers(b):
        idxv, asv, adv, hv, _, sems, _, _ = sets[b]
        pltpu.make_async_copy(as_h.at[idxv.at[0]], asv, sems[0]).wait()
        pltpu.make_async_copy(ad_h.at[idxv.at[1]], adv, sems[1]).wait()
        pltpu.make_async_copy(h_h.at[idxv.at[0]], hv, sems[2]).wait()

    def issue_idx(b, j):
        idxv = sets[b][0]
        sem = sets[b][6]
        pltpu.async_copy(edge_h.at[wid, j], idxv, sem)

    def wait_idx(b, j):
        idxv = sets[b][0]
        sem = sets[b][6]
        pltpu.make_async_copy(edge_h.at[wid, j], idxv, sem).wait()

    _shift_idx = (lax.iota(jnp.int32, 16) + 8) % 16

    def compute_scatter(b):
        idxv, asv, adv, hv, exv, _, _, scs = sets[b]

        def edge(e, ecarry):
            adamv = adv[e]
            a = asv[e] + adamv
            sh = lax.gather(
                adamv, _shift_idx[:, None],
                dimension_numbers=lax.GatherDimensionNumbers(
                    offset_dims=(), collapsed_slice_dims=(0,),
                    start_index_map=(0,)),
                slice_sizes=(1,),
                mode=lax.GatherScatterMode.PROMISE_IN_BOUNDS)
            ex = jnp.exp(_leaky(a) - sh)
            exm = jnp.where(mask8, ex, 0.0)
            exv[e] = exm
            for k in range(HEADS):
                w = exm[k]
                hv[e, pl.ds(16 * k, 16)] = hv[e, pl.ds(16 * k, 16)] * w
            return ecarry

        lax.fori_loop(0, CHUNK, edge, 0)
        pltpu.async_copy(exv, acc_d.at[idxv.at[1]], scs[0], add=True)
        pltpu.async_copy(hv, acc_s.at[idxv.at[1]], scs[1], add=True)

    pltpu.sync_copy(edge_h.at[wid, 0], idxA)
    issue_gathers(0)
    issue_idx(1, 1)

    def wait_scatter(b):
        idxv, _, _, hv, exv, _, _, scs = sets[b]
        pltpu.make_async_copy(exv, acc_d.at[idxv.at[1]], scs[0]).wait()
        pltpu.make_async_copy(hv, acc_s.at[idxv.at[1]], scs[1]).wait()

    def pipe(g, carry):
        j_a = 2 * g
        j_b = 2 * g + 1

        @pl.when(g > 0)
        def _():
            wait_scatter(1)

        @pl.when(j_b < NCHUNK)
        def _():
            wait_idx(1, j_b)
            issue_gathers(1)

        wait_gathers(0)
        compute_scatter(0)

        @pl.when(j_a + 2 < NCHUNK)
        def _():
            issue_idx(0, j_a + 2)

        @pl.when(j_b < NCHUNK)
        def _():
            wait_gathers(1)
            compute_scatter(1)

        @pl.when(j_b + 2 < NCHUNK)
        def _():
            issue_idx(1, j_b + 2)

        @pl.when(j_a + 2 < NCHUNK)
        def _():
            wait_scatter(0)
            wait_idx(0, j_a + 2)
            issue_gathers(0)

        return carry

    lax.fori_loop(0, (NCHUNK + 1) // 2, pipe, 0)
    wait_scatter(0)
    plsc.subcore_barrier()

    def drain(p, carry):
        r0 = rbase + p * CHUNK
        pltpu.sync_copy(acc_s.at[pl.ds(r0, CHUNK)], hA)
        pltpu.sync_copy(hA, out_s.at[c, pl.ds(r0, CHUNK)])
        pltpu.sync_copy(acc_d.at[pl.ds(r0, CHUNK)], exA)
        pltpu.sync_copy(exA, out_d.at[c, pl.ds(r0, CHUNK)])
        return carry

    lax.fori_loop(0, ROWS_PER_TILE // CHUNK, drain, 0)


@functools.partial(
    pl.kernel,
    mesh=plsc.VectorSubcoreMesh(core_axis_name="c", subcore_axis_name="s"),
    compiler_params=pltpu.CompilerParams(use_tc_tiling_on_sc=False),
    out_type=[jax.ShapeDtypeStruct((256, 16), _NF)],
    scratch_types=[
        pltpu.VMEM((2, 128), jnp.int32),
        pltpu.VMEM((128, 16), _NF),
    ],
)
def _tgt_call(scores_h, tgt_h, out_h, tg_v, ot_v):
    c = lax.axis_index("c")
    s = lax.axis_index("s")

    @pl.when(jnp.logical_and(c == 0, s == 0))
    def _():
        pltpu.sync_copy(tgt_h, tg_v)
        for p in range(2):
            pltpu.sync_copy(scores_h.at[tg_v.at[p]], ot_v)
            pltpu.sync_copy(ot_v, out_h.at[pl.ds(128 * p, 128)])


def _expand_att(att):
    a = att.reshape(HEADS * OUTC).astype(jnp.float32)
    m = jnp.repeat(jnp.eye(HEADS, dtype=jnp.float32), OUTC, axis=0)
    p8 = m * a[:, None]
    return jnp.concatenate([p8, jnp.zeros((HIDDEN, 16 - HEADS), jnp.float32)], axis=1)


def kernel(word_embed_matrix, target_mask_list, graph_edge_list, W1, b1,
           Wc0, att_src0, att_dst0, bc0, Wc1, att_src1, att_dst1, bc1, W3, b3):
    edges = jnp.swapaxes(
        graph_edge_list.astype(jnp.int32).reshape(2, NW, NCHUNK, CHUNK),
        0, 1).swapaxes(1, 2)
    r_mat = jnp.repeat(jnp.eye(16, dtype=jnp.float32)[:, :HEADS], OUTC, axis=1)

    h0, as0, adam0 = _inproj_call(
        word_embed_matrix, W1.T, b1.reshape(1, -1), Wc0,
        _expand_att(att_src0), _expand_att(att_dst0))
    s0, d0 = _edge_call(edges, as0, adam0, h0)
    (x1,) = _norm_call(s0, d0, r_mat, bc0.reshape(1, -1))
    h1, as1, adam1 = _proj_call(
        x1, Wc1, _expand_att(att_src1), _expand_att(att_dst1))
    s1, d1 = _edge_call(edges, as1, adam1, h1)
    (scores,) = _normfin_call(s1, d1, r_mat, bc1.reshape(1, -1),
                              W3.reshape(1, -1), b3.reshape(1, 1))
    tgt = target_mask_list.reshape(2, 128).astype(jnp.int32)
    (out2,) = _tgt_call(scores, tgt)
    return out2[:, 0]
